# Initial kernel scaffold; baseline (speedup 1.0000x reference)
import jax
import jax.numpy as jnp
from jax.experimental import pallas as pl


def kernel(target, pred, w_0, b_0, w_1, b_1, w_2, b_2, w_3, b_3, w_4, b_4, w_5, b_5, w_6, b_6, w_7, b_7, w_8, b_8, w_9, b_9):
    raise NotImplementedError("write your pallas kernel here")


# speedup vs baseline: 2.0333x; 2.0333x over previous
import functools

import jax
import jax.numpy as jnp
from jax.experimental import pallas as pl
from jax.experimental.pallas import tpu as pltpu

_VMEM_LIMIT = 48 * 1024 * 1024


def _conv_body(x_hbm, w_ref, b_ref, o_ref, buf, sem, *, th, taps):
    n = pl.program_id(0)
    r = pl.program_id(1)
    nrows = pl.num_programs(1)
    halo = taps - 1
    slot = jax.lax.rem(r, 2)
    cin = buf.shape[3]
    wc = o_ref.shape[2]
    cout = o_ref.shape[3]

    @pl.when(r == 0)
    def _():
        pltpu.make_async_copy(
            x_hbm.at[n, pl.ds(0, th + halo)], buf.at[0], sem.at[0]).start()

    pltpu.make_async_copy(
        x_hbm.at[n, pl.ds(r * th, th + halo)], buf.at[slot], sem.at[slot]).wait()

    @pl.when(r + 1 < nrows)
    def _():
        pltpu.make_async_copy(
            x_hbm.at[n, pl.ds((r + 1) * th, th + halo)],
            buf.at[1 - slot], sem.at[1 - slot]).start()

    acc = jnp.zeros((th * wc, cout), jnp.float32)
    for dy in range(taps):
        slab = buf[slot, dy:dy + th]
        if taps == 1:
            lhs = slab[:, :wc, :]
        else:
            lhs = jnp.concatenate(
                [slab[:, dx:dx + wc, :] for dx in range(taps)], axis=-1)
        acc = acc + jnp.dot(lhs.reshape(th * wc, taps * cin), w_ref[dy],
                            preferred_element_type=jnp.float32)
    y = jnp.maximum(acc + b_ref[...], 0.0)
    o_ref[...] = y.astype(o_ref.dtype).reshape(1, th, wc, cout)


def _conv_call(xp, wk, b, H, Wc, taps, th):
    N, Hp, Wp, Cin = xp.shape
    Cout = wk.shape[-1]
    body = functools.partial(_conv_body, th=th, taps=taps)
    return pl.pallas_call(
        body,
        out_shape=jax.ShapeDtypeStruct((N, H, Wc, Cout), jnp.bfloat16),
        grid_spec=pltpu.PrefetchScalarGridSpec(
            num_scalar_prefetch=0,
            grid=(N, H // th),
            in_specs=[
                pl.BlockSpec(memory_space=pl.ANY),
                pl.BlockSpec((taps, taps * Cin, Cout), lambda n, r: (0, 0, 0)),
                pl.BlockSpec((1, Cout), lambda n, r: (0, 0)),
            ],
            out_specs=pl.BlockSpec((1, th, Wc, Cout), lambda n, r: (n, r, 0, 0)),
            scratch_shapes=[
                pltpu.VMEM((2, th + taps - 1, Wp, Cin), jnp.bfloat16),
                pltpu.SemaphoreType.DMA((2,)),
            ],
        ),
        compiler_params=pltpu.CompilerParams(
            dimension_semantics=("parallel", "arbitrary"),
            vmem_limit_bytes=_VMEM_LIMIT,
        ),
    )(xp, wk, b.reshape(1, Cout))


def _conv3x3_relu(x, w, b):
    N, H, W, Cin = x.shape
    Cout = w.shape[-1]
    Wc = ((W + 7) // 8) * 8
    th = 14 if H == 28 else 28
    xbf = x.astype(jnp.bfloat16)
    if Cin < 8:
        xpad = jnp.pad(xbf, ((0, 0), (1, 1), (1, 1), (0, 0)))
        xi = jnp.concatenate(
            [xpad[:, dy:dy + H, dx:dx + W, :] for dy in range(3) for dx in range(3)],
            axis=-1)
        if Wc != W:
            xi = jnp.pad(xi, ((0, 0), (0, 0), (0, Wc - W), (0, 0)))
        wk = w.reshape(1, 9 * Cin, Cout).astype(jnp.bfloat16)
        out = _conv_call(xi, wk, b, H, Wc, taps=1, th=th)
    else:
        xpd = jnp.pad(xbf, ((0, 0), (1, 1), (1, Wc - W + 7), (0, 0)))
        wk = w.reshape(3, 3 * Cin, Cout).astype(jnp.bfloat16)
        out = _conv_call(xpd, wk, b, H, Wc, taps=3, th=th)
    return out if Wc == W else out[:, :, :W, :]


def _red_body(x_ref, y_ref, o_ref, acc, *, square):
    t = pl.program_id(1)

    @pl.when(t == 0)
    def _():
        acc[...] = jnp.zeros_like(acc)

    d = x_ref[...].astype(jnp.float32) - y_ref[...].astype(jnp.float32)
    v = d * d if square else jnp.abs(d)
    acc[...] += jnp.sum(v.reshape(-1, 8, 128), axis=0)

    @pl.when(t == pl.num_programs(1) - 1)
    def _():
        o_ref[...] = acc[...][None]


def _diff_mean(x, y, *, square):
    n = x.size
    rows = 1024
    tile = rows * 128
    num_tiles = -(-n // tile)
    npar = 2 if num_tiles >= 2 else 1
    nseq = -(-num_tiles // npar)
    num_tiles = npar * nseq
    pad = num_tiles * tile - n
    xf = jnp.pad(x.reshape(-1), (0, pad)).reshape(num_tiles * rows, 128)
    yf = jnp.pad(y.reshape(-1), (0, pad)).reshape(num_tiles * rows, 128)
    part = pl.pallas_call(
        functools.partial(_red_body, square=square),
        out_shape=jax.ShapeDtypeStruct((npar, 8, 128), jnp.float32),
        grid_spec=pltpu.PrefetchScalarGridSpec(
            num_scalar_prefetch=0,
            grid=(npar, nseq),
            in_specs=[
                pl.BlockSpec((rows, 128), lambda p, t, s=nseq: (p * s + t, 0)),
                pl.BlockSpec((rows, 128), lambda p, t, s=nseq: (p * s + t, 0)),
            ],
            out_specs=pl.BlockSpec((1, 8, 128), lambda p, t: (p, 0, 0)),
            scratch_shapes=[pltpu.VMEM((8, 128), jnp.float32)],
        ),
        compiler_params=pltpu.CompilerParams(
            dimension_semantics=("parallel", "arbitrary"),
            vmem_limit_bytes=_VMEM_LIMIT,
        ),
    )(xf, yf)
    return jnp.sum(part) / n


def _ssim(x, y, data_range=255.0, win_size=11, sigma=1.5):
    c1 = (0.01 * data_range) ** 2
    c2 = (0.03 * data_range) ** 2
    coords = jnp.arange(win_size, dtype=jnp.float32) - win_size // 2
    g = jnp.exp(-(coords ** 2) / (2.0 * sigma * sigma))
    g = g / jnp.sum(g)

    H = x.shape[2]
    out = H - win_size + 1
    band = (jnp.zeros((out, H), jnp.float32)
            .at[jnp.arange(out)[:, None], jnp.arange(out)[:, None]
                + jnp.arange(win_size)[None, :]].set(g[None, :]))

    stack = jnp.stack([x, y, x * x, y * y, x * y])
    fh = jnp.einsum('oh,snchw->sncow', band, stack)
    f = jnp.einsum('pw,sncow->sncop', band, fh)
    mu1, mu2, exx, eyy, exy = f[0], f[1], f[2], f[3], f[4]
    s11 = exx - mu1 * mu1
    s22 = eyy - mu2 * mu2
    s12 = exy - mu1 * mu2
    cs = (2.0 * s12 + c2) / (s11 + s22 + c2)
    ssim_map = ((2.0 * mu1 * mu2 + c1) / (mu1 * mu1 + mu2 * mu2 + c1)) * cs
    return jnp.mean(ssim_map)


def _maxpool2(x):
    N, H, W, C = x.shape
    return jnp.max(x.reshape(N, H // 2, 2, W // 2, 2, C), axis=(2, 4))


_BLOCK_LAYERS = ((0, 2), (2, 4), (4, 7), (7, 10))


def _loss(target, pred, params):
    mse = _diff_mean(target, pred, square=True)
    ssim_loss = 1.0 - _ssim(target, pred)

    mean = jnp.array([0.485, 0.456, 0.406], jnp.float32)
    std = jnp.array([0.229, 0.224, 0.225], jnp.float32)
    n = target.shape[0]
    xy = jnp.concatenate([target, pred], axis=0)
    xy = jnp.transpose(xy, (0, 2, 3, 1))
    xy = (xy - mean) / std

    ploss = jnp.float32(0.0)
    h = xy
    for bi, (lo, hi) in enumerate(_BLOCK_LAYERS):
        if bi > 0:
            h = _maxpool2(h)
        for w, b in params[lo:hi]:
            h = _conv3x3_relu(h, w, b)
        ploss = ploss + _diff_mean(h[:n], h[n:], square=False)
    return mse + ssim_loss + 0.1 * ploss


def kernel(target, pred,
           w_0, b_0, w_1, b_1, w_2, b_2, w_3, b_3, w_4, b_4,
           w_5, b_5, w_6, b_6, w_7, b_7, w_8, b_8, w_9, b_9):
    params = [(w_0, b_0), (w_1, b_1), (w_2, b_2), (w_3, b_3), (w_4, b_4),
              (w_5, b_5), (w_6, b_6), (w_7, b_7), (w_8, b_8), (w_9, b_9)]
    return _loss(target, pred, params)


# speedup vs baseline: 2.4616x; 1.2106x over previous
import functools

import jax
import jax.numpy as jnp
from jax.experimental import pallas as pl
from jax.experimental.pallas import tpu as pltpu

_VMEM_LIMIT = 48 * 1024 * 1024


def _conv_body(x_hbm, w_ref, b_ref, o_ref, buf, sem, *, th, taps):
    n = pl.program_id(0)
    r = pl.program_id(1)
    nrows = pl.num_programs(1)
    halo = taps - 1
    slot = jax.lax.rem(r, 2)
    cin = buf.shape[3]
    wc = o_ref.shape[2]
    cout = o_ref.shape[3]

    @pl.when(r == 0)
    def _():
        pltpu.make_async_copy(
            x_hbm.at[n, pl.ds(0, th + halo)], buf.at[0], sem.at[0]).start()

    pltpu.make_async_copy(
        x_hbm.at[n, pl.ds(r * th, th + halo)], buf.at[slot], sem.at[slot]).wait()

    @pl.when(r + 1 < nrows)
    def _():
        pltpu.make_async_copy(
            x_hbm.at[n, pl.ds((r + 1) * th, th + halo)],
            buf.at[1 - slot], sem.at[1 - slot]).start()

    acc = jnp.zeros((th * wc, cout), jnp.float32)
    for dy in range(taps):
        slab = buf[slot, dy:dy + th]
        if taps == 1:
            lhs = slab[:, :wc, :]
        else:
            lhs = jnp.concatenate(
                [slab[:, dx:dx + wc, :] for dx in range(taps)], axis=-1)
        acc = acc + jnp.dot(lhs.reshape(th * wc, taps * cin), w_ref[dy],
                            preferred_element_type=jnp.float32)
    y = jnp.maximum(acc + b_ref[...], 0.0)
    o_ref[...] = y.astype(o_ref.dtype).reshape(1, th, wc, cout)


def _conv_call(xp, wk, b, H, Wc, taps, th):
    N, Hp, Wp, Cin = xp.shape
    Cout = wk.shape[-1]
    body = functools.partial(_conv_body, th=th, taps=taps)
    return pl.pallas_call(
        body,
        out_shape=jax.ShapeDtypeStruct((N, H, Wc, Cout), jnp.bfloat16),
        grid_spec=pltpu.PrefetchScalarGridSpec(
            num_scalar_prefetch=0,
            grid=(N, H // th),
            in_specs=[
                pl.BlockSpec(memory_space=pl.ANY),
                pl.BlockSpec((taps, taps * Cin, Cout), lambda n, r: (0, 0, 0)),
                pl.BlockSpec((1, Cout), lambda n, r: (0, 0)),
            ],
            out_specs=pl.BlockSpec((1, th, Wc, Cout), lambda n, r: (n, r, 0, 0)),
            scratch_shapes=[
                pltpu.VMEM((2, th + taps - 1, Wp, Cin), jnp.bfloat16),
                pltpu.SemaphoreType.DMA((2,)),
            ],
        ),
        compiler_params=pltpu.CompilerParams(
            dimension_semantics=("parallel", "arbitrary"),
            vmem_limit_bytes=_VMEM_LIMIT,
        ),
    )(xp, wk, b.reshape(1, Cout))


def _conv3x3_relu(x, w, b):
    N, H, W, Cin = x.shape
    Cout = w.shape[-1]
    Wc = ((W + 7) // 8) * 8
    th = 14 if H == 28 else 28
    xbf = x.astype(jnp.bfloat16)
    if Cin < 8:
        xpad = jnp.pad(xbf, ((0, 0), (1, 1), (1, 1), (0, 0)))
        xi = jnp.concatenate(
            [xpad[:, dy:dy + H, dx:dx + W, :] for dy in range(3) for dx in range(3)],
            axis=-1)
        if Wc != W:
            xi = jnp.pad(xi, ((0, 0), (0, 0), (0, Wc - W), (0, 0)))
        wk = w.reshape(1, 9 * Cin, Cout).astype(jnp.bfloat16)
        out = _conv_call(xi, wk, b, H, Wc, taps=1, th=th)
    else:
        xpd = jnp.pad(xbf, ((0, 0), (1, 1), (1, Wc - W + 7), (0, 0)))
        wk = w.reshape(3, 3 * Cin, Cout).astype(jnp.bfloat16)
        out = _conv_call(xpd, wk, b, H, Wc, taps=3, th=th)
    return out if Wc == W else out[:, :, :W, :]


def _pair_body(x_hbm, w_ref, b_ref, *refs, th, taps, wc, wtrue, pool):
    if pool:
        pa_ref, pb_ref, d_ref, bufa, bufb, acc, sems = refs
    else:
        d_ref, bufa, bufb, acc, sems = refs
    n = pl.program_id(0)
    nimg = pl.num_programs(0)
    r = pl.program_id(1)
    nrows = pl.num_programs(1)
    halo = taps - 1
    slot = jax.lax.rem(r, 2)
    cin = bufa.shape[3]
    cout = w_ref.shape[2]

    def _start(img, buf, row, s, k):
        pltpu.make_async_copy(
            x_hbm.at[img, pl.ds(row * th, th + halo)], buf.at[s], sems.at[s, k]
        ).start()

    @pl.when(r == 0)
    def _():
        acc[...] = jnp.zeros_like(acc)
        _start(n, bufa, 0, 0, 0)
        _start(n + nimg, bufb, 0, 0, 1)

    pltpu.make_async_copy(
        x_hbm.at[n, pl.ds(r * th, th + halo)], bufa.at[slot], sems.at[slot, 0]).wait()
    pltpu.make_async_copy(
        x_hbm.at[n + nimg, pl.ds(r * th, th + halo)], bufb.at[slot], sems.at[slot, 1]).wait()

    @pl.when(r + 1 < nrows)
    def _():
        _start(n, bufa, r + 1, 1 - slot, 0)
        _start(n + nimg, bufb, r + 1, 1 - slot, 1)

    def _conv_one(buf):
        a = jnp.zeros((th * wc, cout), jnp.float32)
        for dy in range(taps):
            slab = buf[slot, dy:dy + th]
            if taps == 1:
                lhs = slab[:, :wc, :]
            else:
                lhs = jnp.concatenate(
                    [slab[:, dx:dx + wc, :] for dx in range(taps)], axis=-1)
            a = a + jnp.dot(lhs.reshape(th * wc, taps * cin), w_ref[dy],
                            preferred_element_type=jnp.float32)
        return jnp.maximum(a + b_ref[...], 0.0).astype(jnp.bfloat16)

    a16 = _conv_one(bufa)
    b16 = _conv_one(bufb)
    d = jnp.abs(a16.astype(jnp.float32) - b16.astype(jnp.float32))
    if wtrue != wc:
        col = jax.lax.broadcasted_iota(jnp.int32, (th, wc, cout), 1)
        d = jnp.where(col < wtrue, d.reshape(th, wc, cout), 0.0).reshape(th * wc, cout)
    acc[...] += jnp.sum(d, axis=0, keepdims=True)

    if pool:
        pa_ref[...] = a16.reshape(1, th, wc, cout)
        pb_ref[...] = b16.reshape(1, th, wc, cout)

    @pl.when(r == nrows - 1)
    def _():
        d_ref[...] = acc[...][None]


def _conv_pair_call(xp, wk, b, H, Wc, W, taps, th, pool):
    N2, Hp, Wp, Cin = xp.shape
    nimg = N2 // 2
    Cout = wk.shape[-1]
    body = functools.partial(_pair_body, th=th, taps=taps, wc=Wc, wtrue=W, pool=pool)
    out_shape = [jax.ShapeDtypeStruct((nimg, 1, Cout), jnp.float32)]
    out_specs = [pl.BlockSpec((1, 1, Cout), lambda n, r: (n, 0, 0))]
    if pool:
        pshape = jax.ShapeDtypeStruct((nimg, H, Wc, Cout), jnp.bfloat16)
        pspec = pl.BlockSpec((1, th, Wc, Cout), lambda n, r: (n, r, 0, 0))
        out_shape = [pshape, pshape] + out_shape
        out_specs = [pspec, pspec] + out_specs
    res = pl.pallas_call(
        body,
        out_shape=tuple(out_shape),
        grid_spec=pltpu.PrefetchScalarGridSpec(
            num_scalar_prefetch=0,
            grid=(nimg, H // th),
            in_specs=[
                pl.BlockSpec(memory_space=pl.ANY),
                pl.BlockSpec((taps, taps * Cin, Cout), lambda n, r: (0, 0, 0)),
                pl.BlockSpec((1, Cout), lambda n, r: (0, 0)),
            ],
            out_specs=tuple(out_specs),
            scratch_shapes=[
                pltpu.VMEM((2, th + taps - 1, Wp, Cin), jnp.bfloat16),
                pltpu.VMEM((2, th + taps - 1, Wp, Cin), jnp.bfloat16),
                pltpu.VMEM((1, Cout), jnp.float32),
                pltpu.SemaphoreType.DMA((2, 2)),
            ],
        ),
        compiler_params=pltpu.CompilerParams(
            dimension_semantics=("parallel", "arbitrary"),
            vmem_limit_bytes=_VMEM_LIMIT,
        ),
    )(xp, wk, b.reshape(1, Cout))
    if pool:
        return res[0], res[1], jnp.sum(res[2])
    return None, None, jnp.sum(res[0])


def _conv3x3_pair(x, w, b, pool):
    N2, H, W, Cin = x.shape
    Cout = w.shape[-1]
    Wc = ((W + 7) // 8) * 8
    th = 14 if H == 28 else 28
    xpd = jnp.pad(x.astype(jnp.bfloat16), ((0, 0), (1, 1), (1, Wc - W + 7), (0, 0)))
    wk = w.reshape(3, 3 * Cin, Cout).astype(jnp.bfloat16)
    return _conv_pair_call(xpd, wk, b, H, Wc, W, taps=3, th=th, pool=pool)


def _red_body(x_ref, y_ref, o_ref, acc, *, square):
    t = pl.program_id(1)

    @pl.when(t == 0)
    def _():
        acc[...] = jnp.zeros_like(acc)

    d = x_ref[...].astype(jnp.float32) - y_ref[...].astype(jnp.float32)
    v = d * d if square else jnp.abs(d)
    acc[...] += jnp.sum(v.reshape(-1, 8, 128), axis=0)

    @pl.when(t == pl.num_programs(1) - 1)
    def _():
        o_ref[...] = acc[...][None]


def _diff_mean(x, y, *, square):
    n = x.size
    rows = 1024
    tile = rows * 128
    num_tiles = -(-n // tile)
    npar = 2 if num_tiles >= 2 else 1
    nseq = -(-num_tiles // npar)
    num_tiles = npar * nseq
    pad = num_tiles * tile - n
    xf = jnp.pad(x.reshape(-1), (0, pad)).reshape(num_tiles * rows, 128)
    yf = jnp.pad(y.reshape(-1), (0, pad)).reshape(num_tiles * rows, 128)
    part = pl.pallas_call(
        functools.partial(_red_body, square=square),
        out_shape=jax.ShapeDtypeStruct((npar, 8, 128), jnp.float32),
        grid_spec=pltpu.PrefetchScalarGridSpec(
            num_scalar_prefetch=0,
            grid=(npar, nseq),
            in_specs=[
                pl.BlockSpec((rows, 128), lambda p, t, s=nseq: (p * s + t, 0)),
                pl.BlockSpec((rows, 128), lambda p, t, s=nseq: (p * s + t, 0)),
            ],
            out_specs=pl.BlockSpec((1, 8, 128), lambda p, t: (p, 0, 0)),
            scratch_shapes=[pltpu.VMEM((8, 128), jnp.float32)],
        ),
        compiler_params=pltpu.CompilerParams(
            dimension_semantics=("parallel", "arbitrary"),
            vmem_limit_bytes=_VMEM_LIMIT,
        ),
    )(xf, yf)
    return jnp.sum(part) / n


def _ssim(x, y, data_range=255.0, win_size=11, sigma=1.5):
    c1 = (0.01 * data_range) ** 2
    c2 = (0.03 * data_range) ** 2
    coords = jnp.arange(win_size, dtype=jnp.float32) - win_size // 2
    g = jnp.exp(-(coords ** 2) / (2.0 * sigma * sigma))
    g = g / jnp.sum(g)

    H = x.shape[2]
    out = H - win_size + 1
    band = (jnp.zeros((out, H), jnp.float32)
            .at[jnp.arange(out)[:, None], jnp.arange(out)[:, None]
                + jnp.arange(win_size)[None, :]].set(g[None, :]))

    stack = jnp.stack([x, y, x * x, y * y, x * y])
    fh = jnp.einsum('oh,snchw->sncow', band, stack)
    f = jnp.einsum('pw,sncow->sncop', band, fh)
    mu1, mu2, exx, eyy, exy = f[0], f[1], f[2], f[3], f[4]
    s11 = exx - mu1 * mu1
    s22 = eyy - mu2 * mu2
    s12 = exy - mu1 * mu2
    cs = (2.0 * s12 + c2) / (s11 + s22 + c2)
    ssim_map = ((2.0 * mu1 * mu2 + c1) / (mu1 * mu1 + mu2 * mu2 + c1)) * cs
    return jnp.mean(ssim_map)


def _maxpool2(x):
    N, H, W, C = x.shape
    return jnp.max(x.reshape(N, H // 2, 2, W // 2, 2, C), axis=(2, 4))


_BLOCK_LAYERS = ((0, 2), (2, 4), (4, 7), (7, 10))


def _loss(target, pred, params):
    mse = _diff_mean(target, pred, square=True)
    ssim_loss = 1.0 - _ssim(target, pred)

    mean = jnp.array([0.485, 0.456, 0.406], jnp.float32)
    std = jnp.array([0.229, 0.224, 0.225], jnp.float32)
    n = target.shape[0]
    xy = jnp.concatenate([target, pred], axis=0)
    xy = jnp.transpose(xy, (0, 2, 3, 1))
    xy = (xy - mean) / std

    ploss = jnp.float32(0.0)
    h = xy
    for bi, (lo, hi) in enumerate(_BLOCK_LAYERS):
        for w, b in params[lo:hi - 1]:
            h = _conv3x3_relu(h, w, b)
        w, b = params[hi - 1]
        fh, fw, fc = h.shape[1], h.shape[2], w.shape[-1]
        pa, pb, dsum = _conv3x3_pair(h, w, b, pool=(bi < 3))
        ploss = ploss + dsum / (n * fh * fw * fc)
        if bi < 3:
            h = _maxpool2(jnp.concatenate([pa, pb], axis=0))
    return mse + ssim_loss + 0.1 * ploss


def kernel(target, pred,
           w_0, b_0, w_1, b_1, w_2, b_2, w_3, b_3, w_4, b_4,
           w_5, b_5, w_6, b_6, w_7, b_7, w_8, b_8, w_9, b_9):
    params = [(w_0, b_0), (w_1, b_1), (w_2, b_2), (w_3, b_3), (w_4, b_4),
              (w_5, b_5), (w_6, b_6), (w_7, b_7), (w_8, b_8), (w_9, b_9)]
    return _loss(target, pred, params)
